# baseline (speedup 1.0000x reference)
import functools

import jax
import jax.numpy as jnp
from jax.experimental import pallas as pl

_BLK = 6400


def _rowmax_body(x_ref, o_ref, *, cols):
    i = pl.program_id(0)

    @pl.when(i == 0)
    def _init():
        o_ref[...] = jnp.full(o_ref.shape, -jnp.inf, o_ref.dtype)

    x = x_ref[...]
    col = i * _BLK + jax.lax.broadcasted_iota(jnp.int32, x.shape, 1)
    x = jnp.where(col < cols, x, -jnp.inf)
    o_ref[...] = jnp.maximum(o_ref[...], jnp.max(x, axis=-1, keepdims=True))


def kernel(X):
    rows, cols = X.shape
    grid = pl.cdiv(cols, _BLK)
    out = pl.pallas_call(
        functools.partial(_rowmax_body, cols=cols),
        grid=(grid,),
        in_specs=[pl.BlockSpec((rows, _BLK), lambda i: (0, i))],
        out_specs=pl.BlockSpec((rows, 1), lambda i: (0, 0)),
        out_shape=jax.ShapeDtypeStruct((rows, 1), X.dtype),
    )(X)
    return out.reshape(rows)
